# baseline (speedup 1.0000x reference)
import functools

import jax
import jax.numpy as jnp
from jax import lax
from jax.experimental import pallas as pl
from jax.experimental.pallas import tpu as pltpu
from jax.experimental.pallas import tpu_sc as plsc

HIDDEN_DIM = 64
OUTPUT_DIM = 64
_ROWS = 1000000

_NC = 2
_NS = 16
_NW = _NC * _NS
_CHUNK = 128
_L = 16

_RBLK = 4096
_HBLK = _RBLK // 2
_NBLK = pl.cdiv(_ROWS, _RBLK)
_PROWS = _NBLK * _HBLK


def _repack_body(xt_ref, wt_ref, o_ref):
    dn = (((0,), (0,)), ((), ()))
    o_ref[:, 0:OUTPUT_DIM] = lax.dot_general(
        xt_ref[:, 0:_HBLK], wt_ref[...], dn, preferred_element_type=jnp.float32
    )
    o_ref[:, OUTPUT_DIM:] = lax.dot_general(
        xt_ref[:, _HBLK:], wt_ref[...], dn, preferred_element_type=jnp.float32
    )


def _make_repack():
    return pl.pallas_call(
        _repack_body,
        grid=(_NBLK,),
        in_specs=[
            pl.BlockSpec((HIDDEN_DIM, _RBLK), lambda i: (0, i)),
            pl.BlockSpec((HIDDEN_DIM, OUTPUT_DIM), lambda i: (0, 0)),
        ],
        out_specs=pl.BlockSpec((_HBLK, 2 * OUTPUT_DIM), lambda i: (i, 0)),
        out_shape=jax.ShapeDtypeStruct((_PROWS, 2 * OUTPUT_DIM), jnp.float32),
    )


def _make_sc_gather(B):
    b_per_w = B // _NW
    n_chunks = b_per_w // _CHUNK
    mesh = plsc.VectorSubcoreMesh(core_axis_name="c", subcore_axis_name="s")

    @functools.partial(
        pl.kernel,
        mesh=mesh,
        out_type=jax.ShapeDtypeStruct((OUTPUT_DIM, B), jnp.float32),
        scratch_types=[
            pltpu.VMEM((b_per_w,), jnp.int32),
            pltpu.VMEM((OUTPUT_DIM,), jnp.float32),
            pltpu.VMEM((_CHUNK,), jnp.int32),
            pltpu.VMEM((_CHUNK, 2 * OUTPUT_DIM), jnp.float32),
            pltpu.VMEM((OUTPUT_DIM, _CHUNK), jnp.float32),
            pltpu.SemaphoreType.DMA,
        ],
        compiler_params=pltpu.CompilerParams(needs_layout_passes=False),
    )
    def gather_kernel(table_p_hbm, idx_hbm, b_hbm, out_hbm, idx_v, b_v,
                      pid_v, rows_v, ft_v, gsem):
        wid = lax.axis_index("s") * _NC + lax.axis_index("c")
        base = wid * b_per_w
        lanes = lax.iota(jnp.int32, _L)
        pltpu.sync_copy(idx_hbm.at[pl.ds(base, b_per_w)], idx_v)
        pltpu.sync_copy(b_hbm, b_v)
        for c in range(n_chunks):
            coff = c * _CHUNK
            for g in range(_CHUNK // _L):
                u = idx_v[pl.ds(coff + g * _L, _L)]
                pid_v[pl.ds(g * _L, _L)] = (
                    jnp.right_shift(u, 12) * _HBLK
                    + jnp.bitwise_and(u, _HBLK - 1)
                )
            pltpu.async_copy(table_p_hbm.at[pid_v], rows_v, gsem).wait()

            def row_body(r, carry):
                rvec = jnp.full((_L,), r, jnp.int32)
                uvec = plsc.load_gather(idx_v, [rvec + coff])
                off = (
                    jnp.bitwise_and(jnp.right_shift(uvec, 11), 1) * OUTPUT_DIM
                    + lanes
                )
                for j in range(OUTPUT_DIM // _L):
                    val = plsc.load_gather(rows_v, [rvec, off + j * _L])
                    val = val + b_v[pl.ds(j * _L, _L)]
                    plsc.store_scatter(ft_v, [lanes + j * _L, rvec], val)
                return carry

            lax.fori_loop(0, _CHUNK, row_body, 0, unroll=8)
            pltpu.sync_copy(
                ft_v,
                out_hbm.at[
                    :,
                    pl.ds(pl.multiple_of(base + coff, _CHUNK), _CHUNK),
                ],
            )

    return gather_kernel


@jax.jit
def kernel(user_id, embed_table, W, b):
    B = user_id.shape[0]
    idx = user_id.reshape((B,)).astype(jnp.int32)
    table_t = embed_table.T
    table_p = _make_repack()(table_t, W.T)
    out_t = _make_sc_gather(B)(table_p, idx, b)
    return out_t.T.reshape((B, 1, OUTPUT_DIM))


# speedup vs baseline: 1.0080x; 1.0080x over previous
import functools

import jax
import jax.numpy as jnp
from jax import lax
from jax.experimental import pallas as pl
from jax.experimental.pallas import tpu as pltpu
from jax.experimental.pallas import tpu_sc as plsc

HIDDEN_DIM = 64
OUTPUT_DIM = 64
_ROWS = 1000000

_NC = 2
_NS = 16
_NW = _NC * _NS
_CHUNK = 128
_L = 16

_RBLK = 4096
_HBLK = _RBLK // 2
_NBLK = pl.cdiv(_ROWS, _RBLK)
_PROWS = _NBLK * _HBLK


def _repack_body(xt_ref, wt_ref, o_ref):
    dn = (((0,), (0,)), ((), ()))
    o_ref[:, 0:OUTPUT_DIM] = lax.dot_general(
        xt_ref[:, 0:_HBLK], wt_ref[...], dn, preferred_element_type=jnp.float32
    )
    o_ref[:, OUTPUT_DIM:] = lax.dot_general(
        xt_ref[:, _HBLK:], wt_ref[...], dn, preferred_element_type=jnp.float32
    )


def _make_repack():
    return pl.pallas_call(
        _repack_body,
        grid=(_NBLK,),
        in_specs=[
            pl.BlockSpec((HIDDEN_DIM, _RBLK), lambda i: (0, i)),
            pl.BlockSpec((HIDDEN_DIM, OUTPUT_DIM), lambda i: (0, 0)),
        ],
        out_specs=pl.BlockSpec((_HBLK, 2 * OUTPUT_DIM), lambda i: (i, 0)),
        out_shape=jax.ShapeDtypeStruct((_PROWS, 2 * OUTPUT_DIM), jnp.float32),
        compiler_params=pltpu.CompilerParams(
            dimension_semantics=("arbitrary",),
            fuse_transposed_lhs_in_matmul=True,
        ),
    )


def _make_sc_gather(B):
    b_per_w = B // _NW
    n_chunks = b_per_w // _CHUNK
    mesh = plsc.VectorSubcoreMesh(core_axis_name="c", subcore_axis_name="s")

    @functools.partial(
        pl.kernel,
        mesh=mesh,
        out_type=jax.ShapeDtypeStruct((OUTPUT_DIM, B), jnp.float32),
        scratch_types=[
            pltpu.VMEM((b_per_w,), jnp.int32),
            pltpu.VMEM((OUTPUT_DIM,), jnp.float32),
            pltpu.VMEM((_CHUNK,), jnp.int32),
            pltpu.VMEM((_CHUNK, 2 * OUTPUT_DIM), jnp.float32),
            pltpu.VMEM((OUTPUT_DIM, _CHUNK), jnp.float32),
            pltpu.SemaphoreType.DMA,
        ],
        compiler_params=pltpu.CompilerParams(needs_layout_passes=False),
    )
    def gather_kernel(table_p_hbm, idx_hbm, b_hbm, out_hbm, idx_v, b_v,
                      pid_v, rows_v, ft_v, gsem):
        wid = lax.axis_index("s") * _NC + lax.axis_index("c")
        base = wid * b_per_w
        lanes = lax.iota(jnp.int32, _L)
        pltpu.sync_copy(idx_hbm.at[pl.ds(base, b_per_w)], idx_v)
        pltpu.sync_copy(b_hbm, b_v)
        for c in range(n_chunks):
            coff = c * _CHUNK
            for g in range(_CHUNK // _L):
                u = idx_v[pl.ds(coff + g * _L, _L)]
                pid_v[pl.ds(g * _L, _L)] = (
                    jnp.right_shift(u, 12) * _HBLK
                    + jnp.bitwise_and(u, _HBLK - 1)
                )
            pltpu.async_copy(table_p_hbm.at[pid_v], rows_v, gsem).wait()

            def row_body(r, carry):
                rvec = jnp.full((_L,), r, jnp.int32)
                uvec = plsc.load_gather(idx_v, [rvec + coff])
                off = (
                    jnp.bitwise_and(jnp.right_shift(uvec, 11), 1) * OUTPUT_DIM
                    + lanes
                )
                for j in range(OUTPUT_DIM // _L):
                    val = plsc.load_gather(rows_v, [rvec, off + j * _L])
                    val = val + b_v[pl.ds(j * _L, _L)]
                    plsc.store_scatter(ft_v, [lanes + j * _L, rvec], val)
                return carry

            lax.fori_loop(0, _CHUNK, row_body, 0, unroll=8)
            pltpu.sync_copy(
                ft_v,
                out_hbm.at[
                    :,
                    pl.ds(pl.multiple_of(base + coff, _CHUNK), _CHUNK),
                ],
            )

    return gather_kernel


@jax.jit
def kernel(user_id, embed_table, W, b):
    B = user_id.shape[0]
    idx = user_id.reshape((B,)).astype(jnp.int32)
    table_t = embed_table.T
    table_p = _make_repack()(table_t, W.T)
    out_t = _make_sc_gather(B)(table_p, idx, b)
    return out_t.T.reshape((B, 1, OUTPUT_DIM))


# speedup vs baseline: 1.2222x; 1.2125x over previous
import functools

import jax
import jax.numpy as jnp
from jax import lax
from jax.experimental import pallas as pl
from jax.experimental.pallas import tpu as pltpu
from jax.experimental.pallas import tpu_sc as plsc

HIDDEN_DIM = 64
OUTPUT_DIM = 64
_ROWS = 1000000

_NC = 2
_NS = 16
_NW = _NC * _NS
_CHUNK = 128
_L = 16

_RBLK = 8192
_HBLK = _RBLK // 2
_NBLK = pl.cdiv(_ROWS, _RBLK)
_PROWS = _NBLK * _HBLK
_RSHIFT = _RBLK.bit_length() - 1


def _repack_body(xt_ref, wt_ref, o_ref):
    dn = (((0,), (0,)), ((), ()))
    o_ref[:, 0:OUTPUT_DIM] = lax.dot_general(
        xt_ref[:, 0:_HBLK], wt_ref[...], dn, preferred_element_type=jnp.float32
    )
    o_ref[:, OUTPUT_DIM:] = lax.dot_general(
        xt_ref[:, _HBLK:], wt_ref[...], dn, preferred_element_type=jnp.float32
    )


def _make_repack():
    return pl.pallas_call(
        _repack_body,
        grid=(_NBLK,),
        in_specs=[
            pl.BlockSpec((HIDDEN_DIM, _RBLK), lambda i: (0, i)),
            pl.BlockSpec((HIDDEN_DIM, OUTPUT_DIM), lambda i: (0, 0)),
        ],
        out_specs=pl.BlockSpec((_HBLK, 2 * OUTPUT_DIM), lambda i: (i, 0)),
        out_shape=jax.ShapeDtypeStruct((_PROWS, 2 * OUTPUT_DIM), jnp.float32),
        compiler_params=pltpu.CompilerParams(
            dimension_semantics=("arbitrary",),
            fuse_transposed_lhs_in_matmul=True,
        ),
    )


def _make_sc_gather(B):
    b_per_w = B // _NW
    n_chunks = b_per_w // _CHUNK
    mesh = plsc.VectorSubcoreMesh(core_axis_name="c", subcore_axis_name="s")

    @functools.partial(
        pl.kernel,
        mesh=mesh,
        out_type=jax.ShapeDtypeStruct((OUTPUT_DIM, B), jnp.float32),
        scratch_types=[
            pltpu.VMEM((b_per_w,), jnp.int32),
            pltpu.VMEM((OUTPUT_DIM,), jnp.float32),
            pltpu.VMEM((_CHUNK,), jnp.int32),
            pltpu.VMEM((_CHUNK, 2 * OUTPUT_DIM), jnp.float32),
            pltpu.VMEM((OUTPUT_DIM, _CHUNK), jnp.float32),
            pltpu.SemaphoreType.DMA,
        ],
        compiler_params=pltpu.CompilerParams(needs_layout_passes=False),
    )
    def gather_kernel(table_p_hbm, idx_hbm, b_hbm, out_hbm, idx_v, b_v,
                      pid_v, rows_v, ft_v, gsem):
        wid = lax.axis_index("s") * _NC + lax.axis_index("c")
        base = wid * b_per_w
        lanes = lax.iota(jnp.int32, _L)
        pltpu.sync_copy(idx_hbm.at[pl.ds(base, b_per_w)], idx_v)
        pltpu.sync_copy(b_hbm, b_v)
        for c in range(n_chunks):
            coff = c * _CHUNK
            for g in range(_CHUNK // _L):
                u = idx_v[pl.ds(coff + g * _L, _L)]
                pid_v[pl.ds(g * _L, _L)] = (
                    jnp.right_shift(u, _RSHIFT) * _HBLK
                    + jnp.bitwise_and(u, _HBLK - 1)
                )
            pltpu.async_copy(table_p_hbm.at[pid_v], rows_v, gsem).wait()

            def row_body(r, carry):
                rvec = jnp.full((_L,), r, jnp.int32)
                uvec = plsc.load_gather(idx_v, [rvec + coff])
                off = (
                    jnp.bitwise_and(jnp.right_shift(uvec, _RSHIFT - 1), 1)
                    * OUTPUT_DIM
                    + lanes
                )
                for j in range(OUTPUT_DIM // _L):
                    val = plsc.load_gather(rows_v, [rvec, off + j * _L])
                    val = val + b_v[pl.ds(j * _L, _L)]
                    plsc.store_scatter(ft_v, [lanes + j * _L, rvec], val)
                return carry

            lax.fori_loop(0, _CHUNK, row_body, 0, unroll=8)
            pltpu.sync_copy(
                ft_v,
                out_hbm.at[
                    :,
                    pl.ds(pl.multiple_of(base + coff, _CHUNK), _CHUNK),
                ],
            )

    return gather_kernel


@jax.jit
def kernel(user_id, embed_table, W, b):
    B = user_id.shape[0]
    idx = user_id.reshape((B,)).astype(jnp.int32)
    table_t = embed_table.T
    table_p = _make_repack()(table_t, W.T)
    out_t = _make_sc_gather(B)(table_p, idx, b)
    return out_t.T.reshape((B, 1, OUTPUT_DIM))


# speedup vs baseline: 1.3775x; 1.1270x over previous
import functools

import jax
import jax.numpy as jnp
from jax import lax
from jax.experimental import pallas as pl
from jax.experimental.pallas import tpu as pltpu
from jax.experimental.pallas import tpu_sc as plsc

HIDDEN_DIM = 64
OUTPUT_DIM = 64
_ROWS = 1000000

_NC = 2
_NS = 16
_NW = _NC * _NS
_CHUNK = 128
_L = 16

_RBLK = 16384
_HBLK = _RBLK // 2
_NBLK = pl.cdiv(_ROWS, _RBLK)
_PROWS = _NBLK * _HBLK
_RSHIFT = _RBLK.bit_length() - 1


def _repack_body(xt_ref, wt_ref, o_ref):
    dn = (((0,), (0,)), ((), ()))
    o_ref[:, 0:OUTPUT_DIM] = lax.dot_general(
        xt_ref[:, 0:_HBLK], wt_ref[...], dn, preferred_element_type=jnp.float32
    )
    o_ref[:, OUTPUT_DIM:] = lax.dot_general(
        xt_ref[:, _HBLK:], wt_ref[...], dn, preferred_element_type=jnp.float32
    )


def _make_repack():
    return pl.pallas_call(
        _repack_body,
        grid=(_NBLK,),
        in_specs=[
            pl.BlockSpec((HIDDEN_DIM, _RBLK), lambda i: (0, i)),
            pl.BlockSpec((HIDDEN_DIM, OUTPUT_DIM), lambda i: (0, 0)),
        ],
        out_specs=pl.BlockSpec((_HBLK, 2 * OUTPUT_DIM), lambda i: (i, 0)),
        out_shape=jax.ShapeDtypeStruct((_PROWS, 2 * OUTPUT_DIM), jnp.float32),
        compiler_params=pltpu.CompilerParams(
            dimension_semantics=("arbitrary",),
            fuse_transposed_lhs_in_matmul=True,
        ),
    )


def _make_sc_gather(B):
    b_per_w = B // _NW
    n_chunks = b_per_w // _CHUNK
    mesh = plsc.VectorSubcoreMesh(core_axis_name="c", subcore_axis_name="s")

    @functools.partial(
        pl.kernel,
        mesh=mesh,
        out_type=jax.ShapeDtypeStruct((OUTPUT_DIM, B), jnp.float32),
        scratch_types=[
            pltpu.VMEM((b_per_w,), jnp.int32),
            pltpu.VMEM((OUTPUT_DIM,), jnp.float32),
            pltpu.VMEM((_CHUNK,), jnp.int32),
            pltpu.VMEM((_CHUNK, 2 * OUTPUT_DIM), jnp.float32),
            pltpu.VMEM((OUTPUT_DIM, _CHUNK), jnp.float32),
            pltpu.SemaphoreType.DMA,
        ],
        compiler_params=pltpu.CompilerParams(needs_layout_passes=False),
    )
    def gather_kernel(table_p_hbm, idx_hbm, b_hbm, out_hbm, idx_v, b_v,
                      pid_v, rows_v, ft_v, gsem):
        wid = lax.axis_index("s") * _NC + lax.axis_index("c")
        base = wid * b_per_w
        lanes = lax.iota(jnp.int32, _L)
        pltpu.sync_copy(idx_hbm.at[pl.ds(base, b_per_w)], idx_v)
        pltpu.sync_copy(b_hbm, b_v)
        for c in range(n_chunks):
            coff = c * _CHUNK
            for g in range(_CHUNK // _L):
                u = idx_v[pl.ds(coff + g * _L, _L)]
                pid_v[pl.ds(g * _L, _L)] = (
                    jnp.right_shift(u, _RSHIFT) * _HBLK
                    + jnp.bitwise_and(u, _HBLK - 1)
                )
            pltpu.async_copy(table_p_hbm.at[pid_v], rows_v, gsem).wait()

            def row_body(r, carry):
                rvec = jnp.full((_L,), r, jnp.int32)
                uvec = plsc.load_gather(idx_v, [rvec + coff])
                off = (
                    jnp.bitwise_and(jnp.right_shift(uvec, _RSHIFT - 1), 1)
                    * OUTPUT_DIM
                    + lanes
                )
                for j in range(OUTPUT_DIM // _L):
                    val = plsc.load_gather(rows_v, [rvec, off + j * _L])
                    val = val + b_v[pl.ds(j * _L, _L)]
                    plsc.store_scatter(ft_v, [lanes + j * _L, rvec], val)
                return carry

            lax.fori_loop(0, _CHUNK, row_body, 0, unroll=8)
            pltpu.sync_copy(
                ft_v,
                out_hbm.at[
                    :,
                    pl.ds(pl.multiple_of(base + coff, _CHUNK), _CHUNK),
                ],
            )

    return gather_kernel


@jax.jit
def kernel(user_id, embed_table, W, b):
    B = user_id.shape[0]
    idx = user_id.reshape((B,)).astype(jnp.int32)
    table_t = embed_table.T
    table_p = _make_repack()(table_t, W.T)
    out_t = _make_sc_gather(B)(table_p, idx, b)
    return out_t.T.reshape((B, 1, OUTPUT_DIM))


# speedup vs baseline: 1.4435x; 1.0479x over previous
import functools

import jax
import jax.numpy as jnp
from jax import lax
from jax.experimental import pallas as pl
from jax.experimental.pallas import tpu as pltpu
from jax.experimental.pallas import tpu_sc as plsc

HIDDEN_DIM = 64
OUTPUT_DIM = 64
_ROWS = 1000000

_NC = 2
_NS = 16
_NW = _NC * _NS
_CHUNK = 128
_L = 16

_RBLK = 32768
_HBLK = _RBLK // 2
_NBLK = pl.cdiv(_ROWS, _RBLK)
_PROWS = _NBLK * _HBLK
_RSHIFT = _RBLK.bit_length() - 1


def _repack_body(xt_ref, wt_ref, o_ref):
    dn = (((0,), (0,)), ((), ()))
    o_ref[:, 0:OUTPUT_DIM] = lax.dot_general(
        xt_ref[:, 0:_HBLK], wt_ref[...], dn, preferred_element_type=jnp.float32
    )
    o_ref[:, OUTPUT_DIM:] = lax.dot_general(
        xt_ref[:, _HBLK:], wt_ref[...], dn, preferred_element_type=jnp.float32
    )


def _make_repack():
    return pl.pallas_call(
        _repack_body,
        grid=(_NBLK,),
        in_specs=[
            pl.BlockSpec((HIDDEN_DIM, _RBLK), lambda i: (0, i)),
            pl.BlockSpec((HIDDEN_DIM, OUTPUT_DIM), lambda i: (0, 0)),
        ],
        out_specs=pl.BlockSpec((_HBLK, 2 * OUTPUT_DIM), lambda i: (i, 0)),
        out_shape=jax.ShapeDtypeStruct((_PROWS, 2 * OUTPUT_DIM), jnp.float32),
        compiler_params=pltpu.CompilerParams(
            dimension_semantics=("arbitrary",),
            fuse_transposed_lhs_in_matmul=True,
        ),
    )


def _make_sc_gather(B):
    b_per_w = B // _NW
    n_chunks = b_per_w // _CHUNK
    mesh = plsc.VectorSubcoreMesh(core_axis_name="c", subcore_axis_name="s")

    @functools.partial(
        pl.kernel,
        mesh=mesh,
        out_type=jax.ShapeDtypeStruct((OUTPUT_DIM, B), jnp.float32),
        scratch_types=[
            pltpu.VMEM((b_per_w,), jnp.int32),
            pltpu.VMEM((OUTPUT_DIM,), jnp.float32),
            pltpu.VMEM((_CHUNK,), jnp.int32),
            pltpu.VMEM((_CHUNK, 2 * OUTPUT_DIM), jnp.float32),
            pltpu.VMEM((OUTPUT_DIM, _CHUNK), jnp.float32),
            pltpu.SemaphoreType.DMA,
        ],
        compiler_params=pltpu.CompilerParams(needs_layout_passes=False),
    )
    def gather_kernel(table_p_hbm, idx_hbm, b_hbm, out_hbm, idx_v, b_v,
                      pid_v, rows_v, ft_v, gsem):
        wid = lax.axis_index("s") * _NC + lax.axis_index("c")
        base = wid * b_per_w
        lanes = lax.iota(jnp.int32, _L)
        pltpu.sync_copy(idx_hbm.at[pl.ds(base, b_per_w)], idx_v)
        pltpu.sync_copy(b_hbm, b_v)
        for c in range(n_chunks):
            coff = c * _CHUNK
            for g in range(_CHUNK // _L):
                u = idx_v[pl.ds(coff + g * _L, _L)]
                pid_v[pl.ds(g * _L, _L)] = (
                    jnp.right_shift(u, _RSHIFT) * _HBLK
                    + jnp.bitwise_and(u, _HBLK - 1)
                )
            pltpu.async_copy(table_p_hbm.at[pid_v], rows_v, gsem).wait()

            def row_body(r, carry):
                rvec = jnp.full((_L,), r, jnp.int32)
                uvec = plsc.load_gather(idx_v, [rvec + coff])
                off = (
                    jnp.bitwise_and(jnp.right_shift(uvec, _RSHIFT - 1), 1)
                    * OUTPUT_DIM
                    + lanes
                )
                for j in range(OUTPUT_DIM // _L):
                    val = plsc.load_gather(rows_v, [rvec, off + j * _L])
                    val = val + b_v[pl.ds(j * _L, _L)]
                    plsc.store_scatter(ft_v, [lanes + j * _L, rvec], val)
                return carry

            lax.fori_loop(0, _CHUNK, row_body, 0, unroll=8)
            pltpu.sync_copy(
                ft_v,
                out_hbm.at[
                    :,
                    pl.ds(pl.multiple_of(base + coff, _CHUNK), _CHUNK),
                ],
            )

    return gather_kernel


@jax.jit
def kernel(user_id, embed_table, W, b):
    B = user_id.shape[0]
    idx = user_id.reshape((B,)).astype(jnp.int32)
    table_t = embed_table.T
    table_p = _make_repack()(table_t, W.T)
    out_t = _make_sc_gather(B)(table_p, idx, b)
    return out_t.T.reshape((B, 1, OUTPUT_DIM))


# speedup vs baseline: 1.4451x; 1.0011x over previous
import functools

import jax
import jax.numpy as jnp
from jax import lax
from jax.experimental import pallas as pl
from jax.experimental.pallas import tpu as pltpu
from jax.experimental.pallas import tpu_sc as plsc

HIDDEN_DIM = 64
OUTPUT_DIM = 64
_ROWS = 1000000

_NC = 2
_NS = 16
_NW = _NC * _NS
_CHUNK = 128
_L = 16

_RBLK = 32768
_HBLK = _RBLK // 2
_NBLK = pl.cdiv(_ROWS, _RBLK)
_PROWS = _NBLK * _HBLK
_RSHIFT = _RBLK.bit_length() - 1


def _repack_body(xt_ref, wt_ref, o_ref):
    dn = (((0,), (0,)), ((), ()))
    o_ref[:, 0:OUTPUT_DIM] = lax.dot_general(
        xt_ref[:, 0:_HBLK], wt_ref[...], dn,
        preferred_element_type=jnp.float32,
        precision=lax.Precision.DEFAULT,
    )
    o_ref[:, OUTPUT_DIM:] = lax.dot_general(
        xt_ref[:, _HBLK:], wt_ref[...], dn,
        preferred_element_type=jnp.float32,
        precision=lax.Precision.DEFAULT,
    )


def _make_repack():
    return pl.pallas_call(
        _repack_body,
        grid=(_NBLK,),
        in_specs=[
            pl.BlockSpec((HIDDEN_DIM, _RBLK), lambda i: (0, i)),
            pl.BlockSpec((HIDDEN_DIM, OUTPUT_DIM), lambda i: (0, 0)),
        ],
        out_specs=pl.BlockSpec((_HBLK, 2 * OUTPUT_DIM), lambda i: (i, 0)),
        out_shape=jax.ShapeDtypeStruct((_PROWS, 2 * OUTPUT_DIM), jnp.float32),
        compiler_params=pltpu.CompilerParams(
            dimension_semantics=("arbitrary",),
            fuse_transposed_lhs_in_matmul=True,
        ),
    )


def _make_sc_gather(B):
    b_per_w = B // _NW
    n_chunks = b_per_w // _CHUNK
    mesh = plsc.VectorSubcoreMesh(core_axis_name="c", subcore_axis_name="s")

    @functools.partial(
        pl.kernel,
        mesh=mesh,
        out_type=jax.ShapeDtypeStruct((OUTPUT_DIM, B), jnp.float32),
        scratch_types=[
            pltpu.VMEM((b_per_w,), jnp.int32),
            pltpu.VMEM((OUTPUT_DIM,), jnp.float32),
            pltpu.VMEM((_CHUNK,), jnp.int32),
            pltpu.VMEM((_CHUNK, 2 * OUTPUT_DIM), jnp.float32),
            pltpu.VMEM((OUTPUT_DIM, _CHUNK), jnp.float32),
            pltpu.SemaphoreType.DMA,
        ],
        compiler_params=pltpu.CompilerParams(needs_layout_passes=False),
    )
    def gather_kernel(table_p_hbm, idx_hbm, b_hbm, out_hbm, idx_v, b_v,
                      pid_v, rows_v, ft_v, gsem):
        wid = lax.axis_index("s") * _NC + lax.axis_index("c")
        base = wid * b_per_w
        lanes = lax.iota(jnp.int32, _L)
        pltpu.sync_copy(idx_hbm.at[pl.ds(base, b_per_w)], idx_v)
        pltpu.sync_copy(b_hbm, b_v)
        for c in range(n_chunks):
            coff = c * _CHUNK
            for g in range(_CHUNK // _L):
                u = idx_v[pl.ds(coff + g * _L, _L)]
                pid_v[pl.ds(g * _L, _L)] = (
                    jnp.right_shift(u, _RSHIFT) * _HBLK
                    + jnp.bitwise_and(u, _HBLK - 1)
                )
            pltpu.async_copy(table_p_hbm.at[pid_v], rows_v, gsem).wait()

            def row_body(r, carry):
                rvec = jnp.full((_L,), r, jnp.int32)
                uvec = plsc.load_gather(idx_v, [rvec + coff])
                off = (
                    jnp.bitwise_and(jnp.right_shift(uvec, _RSHIFT - 1), 1)
                    * OUTPUT_DIM
                    + lanes
                )
                for j in range(OUTPUT_DIM // _L):
                    val = plsc.load_gather(rows_v, [rvec, off + j * _L])
                    val = val + b_v[pl.ds(j * _L, _L)]
                    plsc.store_scatter(ft_v, [lanes + j * _L, rvec], val)
                return carry

            lax.fori_loop(0, _CHUNK, row_body, 0, unroll=8)
            pltpu.sync_copy(
                ft_v,
                out_hbm.at[
                    :,
                    pl.ds(pl.multiple_of(base + coff, _CHUNK), _CHUNK),
                ],
            )

    return gather_kernel


@jax.jit
def kernel(user_id, embed_table, W, b):
    B = user_id.shape[0]
    idx = user_id.reshape((B,)).astype(jnp.int32)
    table_t = embed_table.T
    table_p = _make_repack()(table_t, W.T)
    out_t = _make_sc_gather(B)(table_p, idx, b)
    return out_t.T.reshape((B, 1, OUTPUT_DIM))


# speedup vs baseline: 1.4483x; 1.0022x over previous
import functools

import jax
import jax.numpy as jnp
from jax import lax
from jax.experimental import pallas as pl
from jax.experimental.pallas import tpu as pltpu
from jax.experimental.pallas import tpu_sc as plsc

HIDDEN_DIM = 64
OUTPUT_DIM = 64
_ROWS = 1000000

_NC = 2
_NS = 16
_NW = _NC * _NS
_CHUNK = 128
_L = 16

_RBLK = 32768
_HBLK = _RBLK // 2
_NBLK = pl.cdiv(_ROWS, _RBLK)
_PROWS = _NBLK * _HBLK
_RSHIFT = _RBLK.bit_length() - 1


def _repack_body(xt_ref, wt_ref, o_ref):
    dn = (((0,), (0,)), ((), ()))
    o_ref[:, 0:OUTPUT_DIM] = lax.dot_general(
        xt_ref[:, 0:_HBLK], wt_ref[...], dn,
        preferred_element_type=jnp.float32,
        precision=lax.Precision.DEFAULT,
    )
    o_ref[:, OUTPUT_DIM:] = lax.dot_general(
        xt_ref[:, _HBLK:], wt_ref[...], dn,
        preferred_element_type=jnp.float32,
        precision=lax.Precision.DEFAULT,
    )


def _make_repack():
    return pl.pallas_call(
        _repack_body,
        grid=(_NBLK,),
        in_specs=[
            pl.BlockSpec((HIDDEN_DIM, _RBLK), lambda i: (0, i)),
            pl.BlockSpec((HIDDEN_DIM, OUTPUT_DIM), lambda i: (0, 0)),
        ],
        out_specs=pl.BlockSpec((_HBLK, 2 * OUTPUT_DIM), lambda i: (i, 0)),
        out_shape=jax.ShapeDtypeStruct((_PROWS, 2 * OUTPUT_DIM), jnp.float32),
        compiler_params=pltpu.CompilerParams(
            dimension_semantics=("arbitrary",),
            fuse_transposed_lhs_in_matmul=True,
        ),
    )


def _make_sc_gather(B):
    b_per_w = B // _NW
    n_chunks = b_per_w // _CHUNK
    mesh = plsc.VectorSubcoreMesh(core_axis_name="c", subcore_axis_name="s")

    @functools.partial(
        pl.kernel,
        mesh=mesh,
        out_type=jax.ShapeDtypeStruct((OUTPUT_DIM, B), jnp.float32),
        scratch_types=[
            pltpu.VMEM((b_per_w,), jnp.int32),
            pltpu.VMEM((OUTPUT_DIM,), jnp.float32),
            pltpu.VMEM((2, _CHUNK), jnp.int32),
            pltpu.VMEM((2, _CHUNK, 2 * OUTPUT_DIM), jnp.float32),
            pltpu.VMEM((OUTPUT_DIM, _CHUNK), jnp.float32),
            pltpu.SemaphoreType.DMA,
            pltpu.SemaphoreType.DMA,
        ],
        compiler_params=pltpu.CompilerParams(needs_layout_passes=False),
    )
    def gather_kernel(table_p_hbm, idx_hbm, b_hbm, out_hbm, idx_v, b_v,
                      pid_v, rows2_v, ft_v, gsem0, gsem1):
        wid = lax.axis_index("s") * _NC + lax.axis_index("c")
        base = wid * b_per_w
        lanes = lax.iota(jnp.int32, _L)
        sems = (gsem0, gsem1)
        pltpu.sync_copy(idx_hbm.at[pl.ds(base, b_per_w)], idx_v)
        pltpu.sync_copy(b_hbm, b_v)

        def stage(c):
            slot = c % 2
            coff = c * _CHUNK
            for g in range(_CHUNK // _L):
                u = idx_v[pl.ds(coff + g * _L, _L)]
                pid_v[slot, pl.ds(g * _L, _L)] = (
                    jnp.right_shift(u, _RSHIFT) * _HBLK
                    + jnp.bitwise_and(u, _HBLK - 1)
                )
            pltpu.async_copy(
                table_p_hbm.at[pid_v.at[slot]], rows2_v.at[slot], sems[slot]
            )

        stage(0)
        for c in range(n_chunks):
            slot = c % 2
            coff = c * _CHUNK
            rows_v = rows2_v.at[slot]
            if c + 1 < n_chunks:
                stage(c + 1)
            pltpu.make_async_copy(
                table_p_hbm.at[pid_v.at[slot]], rows2_v.at[slot], sems[slot]
            ).wait()

            def row_body(r, carry):
                rvec = jnp.full((_L,), r, jnp.int32)
                uvec = plsc.load_gather(idx_v, [rvec + coff])
                off = (
                    jnp.bitwise_and(jnp.right_shift(uvec, _RSHIFT - 1), 1)
                    * OUTPUT_DIM
                    + lanes
                )
                for j in range(OUTPUT_DIM // _L):
                    val = plsc.load_gather(rows_v, [rvec, off + j * _L])
                    val = val + b_v[pl.ds(j * _L, _L)]
                    plsc.store_scatter(ft_v, [lanes + j * _L, rvec], val)
                return carry

            lax.fori_loop(0, _CHUNK, row_body, 0, unroll=8)
            pltpu.sync_copy(
                ft_v,
                out_hbm.at[
                    :,
                    pl.ds(pl.multiple_of(base + coff, _CHUNK), _CHUNK),
                ],
            )

    return gather_kernel


@jax.jit
def kernel(user_id, embed_table, W, b):
    B = user_id.shape[0]
    idx = user_id.reshape((B,)).astype(jnp.int32)
    table_t = embed_table.T
    table_p = _make_repack()(table_t, W.T)
    out_t = _make_sc_gather(B)(table_p, idx, b)
    return out_t.T.reshape((B, 1, OUTPUT_DIM))
